# baseline (speedup 1.0000x reference)
import numpy as np
import jax
import jax.numpy as jnp
from jax.experimental import pallas as pl
from jax.experimental.pallas import tpu as pltpu

_PK = 8


def _bilinear_matrix(out_size, in_size):
    scale = in_size / out_size
    idx = np.arange(out_size)
    src = (idx + 0.5) * scale - 0.5
    src = np.clip(src, 0.0, None)
    i0 = np.minimum(np.floor(src).astype(np.int64), in_size - 1)
    i1 = np.minimum(i0 + 1, in_size - 1)
    w1 = (src - i0).astype(np.float32)
    w0 = (1.0 - w1).astype(np.float32)
    A = np.zeros((out_size, in_size), dtype=np.float32)
    A[idx, i0] += w0
    A[idx, i1] += w1
    return A


def _fused_kernel(*refs):
    pk = len(refs) - 9
    x_refs = refs[:pk]
    (s_ref, r_ref, w27_ref, bb_ref, wc_ref, bc_ref, ah_ref, awt_ref,
     out_ref) = refs[pk:]
    _, NCLS, H, W = out_ref.shape
    C = x_refs[0].shape[1]
    Ho = ah_ref.shape[1]
    WL = pk * W
    Wo = W // 2
    WoL = pk * Wo
    OC = w27_ref.shape[0]

    xcat = jnp.concatenate([r[0] for r in x_refs], axis=2)
    xf = xcat.astype(jnp.bfloat16).reshape(C * H, WL)

    U = jax.lax.dot_general(
        xf, s_ref[...], (((1,), (0,)), ((), ())),
        preferred_element_type=jnp.float32).astype(jnp.bfloat16)
    U = U.reshape(C, H, 3 * WoL)

    r_b = jnp.broadcast_to(r_ref[...], (C, 3 * Ho, H))
    rj = []
    for j in range(3):
        rj.append(jax.lax.dot_general(
            r_b, U[:, :, j * WoL:(j + 1) * WoL], (((2,), (1,)), ((0,), (0,))),
            preferred_element_type=jnp.float32).astype(jnp.bfloat16))

    slabs = []
    for i in range(3):
        for j in range(3):
            slabs.append(rj[j][:, i * Ho:(i + 1) * Ho, :])
    P = jnp.concatenate(slabs, axis=0)

    feat = jax.lax.dot_general(
        w27_ref[...], P, (((1,), (0,)), ((), ())),
        preferred_element_type=jnp.float32)
    feat = jnp.maximum(feat + bb_ref[...][:, :, None], 0.0).astype(jnp.bfloat16)

    logits = jax.lax.dot_general(
        wc_ref[...], feat, (((1,), (0,)), ((), ())),
        preferred_element_type=jnp.float32)
    logits = (logits + bc_ref[...][:, :, None]).astype(jnp.bfloat16)

    t = jax.lax.dot_general(
        logits, awt_ref[...], (((2,), (0,)), ((), ())),
        preferred_element_type=jnp.float32)
    t = t.astype(jnp.bfloat16)

    ah_b = jnp.broadcast_to(ah_ref[...], (NCLS, H, Ho))
    for k in range(pk):
        y = jax.lax.dot_general(
            ah_b, t[:, :, k * W:(k + 1) * W], (((2,), (1,)), ((0,), (0,))),
            preferred_element_type=jnp.float32)
        out_ref[k] = y


def kernel(x, backbone_w, backbone_b, cls_w, cls_b):
    N, C, H, W = x.shape
    OC = backbone_w.shape[0]
    NCLS = cls_w.shape[0]
    Ho = (H + 2 - 3) // 2 + 1
    Wo = (W + 2 - 3) // 2 + 1
    pk = next(k for k in (_PK, 4, 2, 1) if k <= N and N % k == 0)
    WoL = pk * Wo

    S = np.zeros((pk * W, 3 * WoL), np.float32)
    for img in range(pk):
        for j in range(3):
            for wo in range(Wo):
                w = 2 * wo + j - 1
                if 0 <= w < W:
                    S[img * W + w, j * WoL + img * Wo + wo] = 1.0
    R = np.zeros((3 * Ho, H), np.float32)
    for i in range(3):
        for ho in range(Ho):
            h = 2 * ho + i - 1
            if 0 <= h < H:
                R[i * Ho + ho, h] = 1.0
    S = jnp.asarray(S, jnp.bfloat16)
    R = jnp.asarray(R, jnp.bfloat16)

    w27 = backbone_w.transpose(0, 2, 3, 1).reshape(OC, 9 * C)
    w27 = w27.astype(jnp.bfloat16)
    bb2 = backbone_b.reshape(OC, 1).astype(jnp.float32)
    wc2 = cls_w.reshape(NCLS, OC).astype(jnp.bfloat16)
    bc2 = cls_b.reshape(NCLS, 1).astype(jnp.float32)
    ah = jnp.asarray(_bilinear_matrix(H, Ho), jnp.bfloat16)
    awt = _bilinear_matrix(W, Wo).T
    awt4 = np.zeros((WoL, pk * W), np.float32)
    for img in range(pk):
        awt4[img * Wo:(img + 1) * Wo, img * W:(img + 1) * W] = awt
    awt4 = jnp.asarray(awt4, jnp.bfloat16)

    x_specs = [
        pl.BlockSpec((1, C, H, W), lambda n, k=k: (pk * n + k, 0, 0, 0))
        for k in range(pk)
    ]
    out = pl.pallas_call(
        _fused_kernel,
        out_shape=jax.ShapeDtypeStruct((N, NCLS, H, W), jnp.float32),
        grid=(N // pk,),
        in_specs=x_specs + [
            pl.BlockSpec((pk * W, 3 * WoL), lambda n: (0, 0)),
            pl.BlockSpec((3 * Ho, H), lambda n: (0, 0)),
            pl.BlockSpec((OC, 9 * C), lambda n: (0, 0)),
            pl.BlockSpec((OC, 1), lambda n: (0, 0)),
            pl.BlockSpec((NCLS, OC), lambda n: (0, 0)),
            pl.BlockSpec((NCLS, 1), lambda n: (0, 0)),
            pl.BlockSpec((H, Ho), lambda n: (0, 0)),
            pl.BlockSpec((WoL, pk * W), lambda n: (0, 0)),
        ],
        out_specs=pl.BlockSpec((pk, NCLS, H, W), lambda n: (n, 0, 0, 0)),
        compiler_params=pltpu.CompilerParams(dimension_semantics=("parallel",)),
    )(*([x] * pk), S, R, w27, bb2, wc2, bc2, ah, awt4)
    return out


# speedup vs baseline: 1.1476x; 1.1476x over previous
import numpy as np
import jax
import jax.numpy as jnp
from jax.experimental import pallas as pl
from jax.experimental.pallas import tpu as pltpu

_PK = 4


def _bilinear_matrix(out_size, in_size):
    scale = in_size / out_size
    idx = np.arange(out_size)
    src = (idx + 0.5) * scale - 0.5
    src = np.clip(src, 0.0, None)
    i0 = np.minimum(np.floor(src).astype(np.int64), in_size - 1)
    i1 = np.minimum(i0 + 1, in_size - 1)
    w1 = (src - i0).astype(np.float32)
    w0 = (1.0 - w1).astype(np.float32)
    A = np.zeros((out_size, in_size), dtype=np.float32)
    A[idx, i0] += w0
    A[idx, i1] += w1
    return A


def _fused_kernel(*refs):
    pk = len(refs) - 9
    x_refs = refs[:pk]
    (s_ref, r_ref, w27_ref, bb_ref, wc_ref, bc_ref, ah_ref, awt_ref,
     out_ref) = refs[pk:]
    _, NCLS, H, W = out_ref.shape
    C = x_refs[0].shape[1]
    Ho = ah_ref.shape[1]
    WL = pk * W
    Wo = W // 2
    WoL = pk * Wo
    OC = w27_ref.shape[0]

    xcat = jnp.concatenate([r[0] for r in x_refs], axis=2)
    xf = xcat.astype(jnp.bfloat16).reshape(C * H, WL)

    U = jax.lax.dot_general(
        xf, s_ref[...], (((1,), (0,)), ((), ())),
        preferred_element_type=jnp.float32).astype(jnp.bfloat16)
    U = U.reshape(C, H, 3 * WoL)

    r_b = jnp.broadcast_to(r_ref[...], (C, 3 * Ho, H))
    rj = []
    for j in range(3):
        rj.append(jax.lax.dot_general(
            r_b, U[:, :, j * WoL:(j + 1) * WoL], (((2,), (1,)), ((0,), (0,))),
            preferred_element_type=jnp.float32).astype(jnp.bfloat16))

    slabs = []
    for i in range(3):
        for j in range(3):
            slabs.append(rj[j][:, i * Ho:(i + 1) * Ho, :])
    P = jnp.concatenate(slabs, axis=0)

    feat = jax.lax.dot_general(
        w27_ref[...], P, (((1,), (0,)), ((), ())),
        preferred_element_type=jnp.float32)
    feat = jnp.maximum(feat + bb_ref[...][:, :, None], 0.0).astype(jnp.bfloat16)

    logits = jax.lax.dot_general(
        wc_ref[...], feat, (((1,), (0,)), ((), ())),
        preferred_element_type=jnp.float32)
    logits = (logits + bc_ref[...][:, :, None]).astype(jnp.bfloat16)

    t = jax.lax.dot_general(
        logits, awt_ref[...], (((2,), (0,)), ((), ())),
        preferred_element_type=jnp.float32)
    t = t.astype(jnp.bfloat16)

    ah_b = jnp.broadcast_to(ah_ref[...], (NCLS, H, Ho))
    for k in range(pk):
        y = jax.lax.dot_general(
            ah_b, t[:, :, k * W:(k + 1) * W], (((2,), (1,)), ((0,), (0,))),
            preferred_element_type=jnp.float32)
        out_ref[k] = y


def kernel(x, backbone_w, backbone_b, cls_w, cls_b):
    N, C, H, W = x.shape
    OC = backbone_w.shape[0]
    NCLS = cls_w.shape[0]
    Ho = (H + 2 - 3) // 2 + 1
    Wo = (W + 2 - 3) // 2 + 1
    pk = next(k for k in (_PK, 4, 2, 1) if k <= N and N % k == 0)
    WoL = pk * Wo

    S = np.zeros((pk * W, 3 * WoL), np.float32)
    for img in range(pk):
        for j in range(3):
            for wo in range(Wo):
                w = 2 * wo + j - 1
                if 0 <= w < W:
                    S[img * W + w, j * WoL + img * Wo + wo] = 1.0
    R = np.zeros((3 * Ho, H), np.float32)
    for i in range(3):
        for ho in range(Ho):
            h = 2 * ho + i - 1
            if 0 <= h < H:
                R[i * Ho + ho, h] = 1.0
    S = jnp.asarray(S, jnp.bfloat16)
    R = jnp.asarray(R, jnp.bfloat16)

    w27 = backbone_w.transpose(0, 2, 3, 1).reshape(OC, 9 * C)
    w27 = w27.astype(jnp.bfloat16)
    bb2 = backbone_b.reshape(OC, 1).astype(jnp.float32)
    wc2 = cls_w.reshape(NCLS, OC).astype(jnp.bfloat16)
    bc2 = cls_b.reshape(NCLS, 1).astype(jnp.float32)
    ah = jnp.asarray(_bilinear_matrix(H, Ho), jnp.bfloat16)
    awt = _bilinear_matrix(W, Wo).T
    awt4 = np.zeros((WoL, pk * W), np.float32)
    for img in range(pk):
        awt4[img * Wo:(img + 1) * Wo, img * W:(img + 1) * W] = awt
    awt4 = jnp.asarray(awt4, jnp.bfloat16)

    x_specs = [
        pl.BlockSpec((1, C, H, W), lambda n, k=k: (pk * n + k, 0, 0, 0))
        for k in range(pk)
    ]
    out = pl.pallas_call(
        _fused_kernel,
        out_shape=jax.ShapeDtypeStruct((N, NCLS, H, W), jnp.float32),
        grid=(N // pk,),
        in_specs=x_specs + [
            pl.BlockSpec((pk * W, 3 * WoL), lambda n: (0, 0)),
            pl.BlockSpec((3 * Ho, H), lambda n: (0, 0)),
            pl.BlockSpec((OC, 9 * C), lambda n: (0, 0)),
            pl.BlockSpec((OC, 1), lambda n: (0, 0)),
            pl.BlockSpec((NCLS, OC), lambda n: (0, 0)),
            pl.BlockSpec((NCLS, 1), lambda n: (0, 0)),
            pl.BlockSpec((H, Ho), lambda n: (0, 0)),
            pl.BlockSpec((WoL, pk * W), lambda n: (0, 0)),
        ],
        out_specs=pl.BlockSpec((pk, NCLS, H, W), lambda n: (n, 0, 0, 0)),
        compiler_params=pltpu.CompilerParams(dimension_semantics=("parallel",)),
    )(*([x] * pk), S, R, w27, bb2, wc2, bc2, ah, awt4)
    return out


# speedup vs baseline: 1.2916x; 1.1254x over previous
import numpy as np
import jax
import jax.numpy as jnp
from jax.experimental import pallas as pl
from jax.experimental.pallas import tpu as pltpu

_PK = 4


def _bilinear_matrix(out_size, in_size):
    scale = in_size / out_size
    idx = np.arange(out_size)
    src = (idx + 0.5) * scale - 0.5
    src = np.clip(src, 0.0, None)
    i0 = np.minimum(np.floor(src).astype(np.int64), in_size - 1)
    i1 = np.minimum(i0 + 1, in_size - 1)
    w1 = (src - i0).astype(np.float32)
    w0 = (1.0 - w1).astype(np.float32)
    A = np.zeros((out_size, in_size), dtype=np.float32)
    A[idx, i0] += w0
    A[idx, i1] += w1
    return A


def _fused_kernel(*refs):
    pk = len(refs) - 9
    x_refs = refs[:pk]
    (s_ref, r_ref, w27_ref, bb_ref, wc_ref, bc_ref, ah_ref, awt_ref,
     out_ref) = refs[pk:]
    _, NCLS, H, W = out_ref.shape
    C = x_refs[0].shape[1]
    Ho = ah_ref.shape[1]
    WL = pk * W
    Wo = W // 2
    WoL = pk * Wo
    OC = w27_ref.shape[0]

    xcat = jnp.concatenate([r[0] for r in x_refs], axis=2)
    xf = xcat.astype(jnp.bfloat16).reshape(C * H, WL)

    U = jax.lax.dot_general(
        xf, s_ref[...], (((1,), (0,)), ((), ())),
        preferred_element_type=jnp.float32).astype(jnp.bfloat16)
    U = U.reshape(C, H, 3 * WoL)

    r_b = jnp.broadcast_to(r_ref[...], (C, 3 * Ho, H))
    rj = []
    for j in range(3):
        rj.append(jax.lax.dot_general(
            r_b, U[:, :, j * WoL:(j + 1) * WoL], (((2,), (1,)), ((0,), (0,))),
            preferred_element_type=jnp.float32).astype(jnp.bfloat16))

    slabs = []
    for i in range(3):
        for j in range(3):
            slabs.append(rj[j][:, i * Ho:(i + 1) * Ho, :])
    P = jnp.concatenate(slabs, axis=0)

    Pb = jnp.swapaxes(P, 0, 1)
    w_b = jnp.broadcast_to(w27_ref[...], (Ho, OC, 9 * C))
    feat = jax.lax.dot_general(
        w_b, Pb, (((2,), (1,)), ((0,), (0,))),
        preferred_element_type=jnp.float32)
    feat = jnp.maximum(feat + bb_ref[...][None], 0.0).astype(jnp.bfloat16)

    logits = jax.lax.dot_general(
        wc_ref[...], feat, (((1,), (1,)), ((), ())),
        preferred_element_type=jnp.float32)
    logits = (logits + bc_ref[...][:, :, None]).astype(jnp.bfloat16)

    t = jax.lax.dot_general(
        logits, awt_ref[...], (((2,), (0,)), ((), ())),
        preferred_element_type=jnp.float32)
    t = t.astype(jnp.bfloat16)

    ah_b = jnp.broadcast_to(ah_ref[...], (NCLS, H, Ho))
    for k in range(pk):
        y = jax.lax.dot_general(
            ah_b, t[:, :, k * W:(k + 1) * W], (((2,), (1,)), ((0,), (0,))),
            preferred_element_type=jnp.float32)
        out_ref[k] = y


def kernel(x, backbone_w, backbone_b, cls_w, cls_b):
    N, C, H, W = x.shape
    OC = backbone_w.shape[0]
    NCLS = cls_w.shape[0]
    Ho = (H + 2 - 3) // 2 + 1
    Wo = (W + 2 - 3) // 2 + 1
    pk = next(k for k in (_PK, 4, 2, 1) if k <= N and N % k == 0)
    WoL = pk * Wo

    S = np.zeros((pk * W, 3 * WoL), np.float32)
    for img in range(pk):
        for j in range(3):
            for wo in range(Wo):
                w = 2 * wo + j - 1
                if 0 <= w < W:
                    S[img * W + w, j * WoL + img * Wo + wo] = 1.0
    R = np.zeros((3 * Ho, H), np.float32)
    for i in range(3):
        for ho in range(Ho):
            h = 2 * ho + i - 1
            if 0 <= h < H:
                R[i * Ho + ho, h] = 1.0
    S = jnp.asarray(S, jnp.bfloat16)
    R = jnp.asarray(R, jnp.bfloat16)

    w27 = backbone_w.transpose(0, 2, 3, 1).reshape(OC, 9 * C)
    w27 = w27.astype(jnp.bfloat16)
    bb2 = backbone_b.reshape(OC, 1).astype(jnp.float32)
    wc2 = cls_w.reshape(NCLS, OC).astype(jnp.bfloat16)
    bc2 = cls_b.reshape(NCLS, 1).astype(jnp.float32)
    ah = jnp.asarray(_bilinear_matrix(H, Ho), jnp.bfloat16)
    awt = _bilinear_matrix(W, Wo).T
    awt4 = np.zeros((WoL, pk * W), np.float32)
    for img in range(pk):
        awt4[img * Wo:(img + 1) * Wo, img * W:(img + 1) * W] = awt
    awt4 = jnp.asarray(awt4, jnp.bfloat16)

    x_specs = [
        pl.BlockSpec((1, C, H, W), lambda n, k=k: (pk * n + k, 0, 0, 0))
        for k in range(pk)
    ]
    out = pl.pallas_call(
        _fused_kernel,
        out_shape=jax.ShapeDtypeStruct((N, NCLS, H, W), jnp.float32),
        grid=(N // pk,),
        in_specs=x_specs + [
            pl.BlockSpec((pk * W, 3 * WoL), lambda n: (0, 0)),
            pl.BlockSpec((3 * Ho, H), lambda n: (0, 0)),
            pl.BlockSpec((OC, 9 * C), lambda n: (0, 0)),
            pl.BlockSpec((OC, 1), lambda n: (0, 0)),
            pl.BlockSpec((NCLS, OC), lambda n: (0, 0)),
            pl.BlockSpec((NCLS, 1), lambda n: (0, 0)),
            pl.BlockSpec((H, Ho), lambda n: (0, 0)),
            pl.BlockSpec((WoL, pk * W), lambda n: (0, 0)),
        ],
        out_specs=pl.BlockSpec((pk, NCLS, H, W), lambda n: (n, 0, 0, 0)),
        compiler_params=pltpu.CompilerParams(dimension_semantics=("parallel",)),
    )(*([x] * pk), S, R, w27, bb2, wc2, bc2, ah, awt4)
    return out
